# baseline (speedup 1.0000x reference)
import functools

import jax
import jax.numpy as jnp
from jax import lax
from jax.experimental import pallas as pl
from jax.experimental.pallas import tpu as pltpu
from jax.experimental.pallas import tpu_sc as plsc

S = 2048
H = 768
F = 2048
E = 8
LANES = 128
TM = 512
NT = S * 2 // TM + E
PADDED = NT * TM
NW = 32
TPW = S // NW
CH = 32


def _cumsum_rows(a):
    n = a.shape[0]
    sh = 1
    while sh < n:
        a = a + jnp.concatenate(
            [jnp.zeros((sh, a.shape[1]), a.dtype), a[: n - sh]], axis=0)
        sh *= 2
    return a


def _cumsum_lanes8(a):
    for sh in (1, 2, 4):
        a = a + jnp.concatenate(
            [jnp.zeros((a.shape[0], sh), a.dtype), a[:, : a.shape[1] - sh]],
            axis=1)
    return a


def _router_body(x_ref, wr_ref, ints_ref, flt_ref):
    x = x_ref[...]
    wr = wr_ref[...]
    logits = jnp.dot(x, wr, preferred_element_type=jnp.float32)
    col = lax.broadcasted_iota(jnp.int32, (S, LANES), 1)
    valid = col < E
    lg = jnp.where(valid, logits, jnp.float32(-1e30))
    m = jnp.max(lg, axis=1, keepdims=True)
    ex = jnp.where(valid, jnp.exp(lg - m), 0.0)
    probs = ex / jnp.sum(ex, axis=1, keepdims=True)

    usage = jnp.sum(probs, axis=0, keepdims=True) * (1.0 / S)
    dd = jnp.where(valid[0:1, :], usage - (1.0 / E), 0.0)
    aux = jnp.sum(dd * dd) * 0.01

    v0 = jnp.max(probs, axis=1, keepdims=True)
    i0 = jnp.min(jnp.where((probs == v0) & valid, col, LANES), axis=1,
                 keepdims=True)
    pm = jnp.where((col == i0) | ~valid, -1.0, probs)
    v1 = jnp.max(pm, axis=1, keepdims=True)
    i1 = jnp.min(jnp.where(pm == v1, col, LANES), axis=1, keepdims=True)
    sn = v0 + v1
    p0 = v0 / sn
    p1 = v1 / sn

    oc = ((col == i0) | (col == i1 + 8)).astype(jnp.float32)
    mask_lo = (col < E).astype(jnp.float32)
    mask_hi = ((col >= 8) & (col < 16)).astype(jnp.float32)
    inc = _cumsum_rows(oc)
    r = inc - oc
    tot = inc[S - 1:S, :]
    z8 = jnp.zeros((1, 8), jnp.float32)
    tot_hi = jnp.concatenate([z8, tot[:, :LANES - 8]], axis=1)
    rank0 = jnp.sum(r * oc * mask_lo, axis=1, keepdims=True)
    rank1 = jnp.sum((r + tot_hi) * oc * mask_hi, axis=1, keepdims=True)
    counts = tot + jnp.concatenate(
        [tot[:, 8:], jnp.zeros((1, 8), jnp.float32)], axis=1)
    pcnt = jnp.floor((counts + (TM - 1)) * (1.0 / TM)) * TM * mask_lo[0:1, :]
    inc_p = _cumsum_lanes8(pcnt)
    off = inc_p - pcnt
    off_hi = jnp.concatenate([z8, off[:, :LANES - 8]], axis=1)
    base0 = jnp.sum(oc * mask_lo * off, axis=1, keepdims=True)
    base1 = jnp.sum(oc * mask_hi * off_hi, axis=1, keepdims=True)
    dest0 = (base0 + rank0).astype(jnp.int32)
    dest1 = (base1 + rank1).astype(jnp.int32)

    start = (lax.broadcasted_iota(jnp.int32, (S, 1), 0) * TM).astype(jnp.float32)
    indmat = ((inc_p <= start) & valid).astype(jnp.float32)
    te = jnp.minimum(jnp.sum(indmat, axis=1, keepdims=True),
                     float(E - 1)).astype(jnp.int32)
    total = inc_p[:, E - 1:E]
    vt = (start < total).astype(jnp.int32)

    ints_ref[...] = jnp.where(
        col == 0, dest0,
        jnp.where(col == 1, dest1,
                  jnp.where(col == 2, te, jnp.where(col == 3, vt, 0))))
    flt_ref[...] = jnp.where(col < 16, p0, jnp.where(col < 32, p1, aux))


def _router_call(x2d, wrp, interpret=False):
    return pl.pallas_call(
        _router_body,
        out_shape=(jax.ShapeDtypeStruct((S, LANES), jnp.int32),
                   jax.ShapeDtypeStruct((S, LANES), jnp.float32)),
        interpret=interpret,
    )(x2d, wrp)


def _ffn_body(te_ref, vd_ref, xs_ref, wg_ref, wu_ref, wd_ref, ys_ref):
    t = pl.program_id(0)

    @pl.when(vd_ref[t] == 1)
    def _():
        xb = xs_ref[...].astype(jnp.bfloat16)
        wg = wg_ref[0].astype(jnp.bfloat16)
        wu = wu_ref[0].astype(jnp.bfloat16)
        g = jnp.dot(xb, wg, preferred_element_type=jnp.float32)
        u = jnp.dot(xb, wu, preferred_element_type=jnp.float32)
        h = ((g * jax.nn.sigmoid(g)) * u).astype(jnp.bfloat16)
        wd = wd_ref[0].astype(jnp.bfloat16)
        ys_ref[...] = jnp.dot(h, wd, preferred_element_type=jnp.float32)


def _ffn_call(te, vd, xs, w_gate, w_up, w_down, interpret=False):
    grid_spec = pltpu.PrefetchScalarGridSpec(
        num_scalar_prefetch=2,
        grid=(NT,),
        in_specs=[
            pl.BlockSpec((TM, H), lambda t, te, vd: (t * vd[t], 0)),
            pl.BlockSpec((1, H, F), lambda t, te, vd: (te[t], 0, 0)),
            pl.BlockSpec((1, H, F), lambda t, te, vd: (te[t], 0, 0)),
            pl.BlockSpec((1, F, H), lambda t, te, vd: (te[t], 0, 0)),
        ],
        out_specs=pl.BlockSpec((TM, H), lambda t, te, vd: (t, 0)),
    )
    return pl.pallas_call(
        _ffn_body,
        grid_spec=grid_spec,
        out_shape=jax.ShapeDtypeStruct((PADDED, H), jnp.float32),
        interpret=interpret,
    )(te, vd, xs, w_gate, w_up, w_down)


def _sc_scatter_call(x2d, dest2):
    mesh = plsc.VectorSubcoreMesh(core_axis_name="c", subcore_axis_name="s")

    @functools.partial(
        pl.kernel,
        out_type=jax.ShapeDtypeStruct((PADDED, H), jnp.float32),
        mesh=mesh,
        scratch_types=[
            pltpu.VMEM((TPW, H), jnp.float32),
            pltpu.VMEM((TPW,), jnp.int32),
            pltpu.VMEM((TPW,), jnp.int32),
            pltpu.SemaphoreType.DMA,
        ],
    )
    def k(x_hbm, d_hbm, xs_hbm, xbuf, i0buf, i1buf, sem):
        wid = lax.axis_index("s") * 2 + lax.axis_index("c")
        base = wid * TPW
        pltpu.sync_copy(x_hbm.at[pl.ds(base, TPW)], xbuf)
        pltpu.sync_copy(d_hbm.at[pl.ds(base, TPW)], i0buf)
        pltpu.sync_copy(d_hbm.at[pl.ds(S + base, TPW)], i1buf)
        pltpu.async_copy(xbuf, xs_hbm.at[i0buf], sem).wait()
        pltpu.async_copy(xbuf, xs_hbm.at[i1buf], sem).wait()

    return k(x2d, dest2)


def _sc_combine_call(ys, dest2, probs0, probs1):
    mesh = plsc.VectorSubcoreMesh(core_axis_name="c", subcore_axis_name="s")

    @functools.partial(
        pl.kernel,
        out_type=jax.ShapeDtypeStruct((S, H), jnp.float32),
        mesh=mesh,
        scratch_types=[
            pltpu.VMEM((CH, H), jnp.float32),
            pltpu.VMEM((CH, H), jnp.float32),
            pltpu.VMEM((CH, H), jnp.float32),
            pltpu.VMEM((CH,), jnp.int32),
            pltpu.VMEM((CH,), jnp.int32),
            pltpu.VMEM((CH * 16,), jnp.float32),
            pltpu.VMEM((CH * 16,), jnp.float32),
            pltpu.SemaphoreType.DMA,
        ],
    )
    def k(ys_hbm, d_hbm, p0_hbm, p1_hbm, out_hbm,
          y0buf, y1buf, obuf, i0buf, i1buf, p0buf, p1buf, sem):
        wid = lax.axis_index("s") * 2 + lax.axis_index("c")
        base = wid * TPW
        for c in range(TPW // CH):
            cb = base + c * CH
            pltpu.sync_copy(d_hbm.at[pl.ds(cb, CH)], i0buf)
            pltpu.sync_copy(d_hbm.at[pl.ds(S + cb, CH)], i1buf)
            pltpu.sync_copy(p0_hbm.at[pl.ds(cb * 16, CH * 16)], p0buf)
            pltpu.sync_copy(p1_hbm.at[pl.ds(cb * 16, CH * 16)], p1buf)
            pltpu.async_copy(ys_hbm.at[i0buf], y0buf, sem).wait()
            pltpu.async_copy(ys_hbm.at[i1buf], y1buf, sem).wait()

            @plsc.parallel_loop(0, CH, unroll=2)
            def _(i):
                pa = p0buf[pl.ds(i * 16, 16)]
                pb = p1buf[pl.ds(i * 16, 16)]
                for j in range(H // 16):
                    sl = pl.ds(j * 16, 16)
                    obuf[i, sl] = y0buf[i, sl] * pa + y1buf[i, sl] * pb

            pltpu.sync_copy(obuf, out_hbm.at[pl.ds(cb, CH)])

    return k(ys, dest2, probs0, probs1)


def kernel(x, w_router, w_gate, w_up, w_down):
    x2d = x.reshape(S, H)
    wrp = jnp.zeros((H, LANES), jnp.float32).at[:, :E].set(w_router.T)
    ints, flts = _router_call(x2d, wrp)
    dflat = jnp.concatenate([ints[:, 0], ints[:, 1]])
    te = ints[:NT, 2]
    vd = ints[:NT, 3]
    aux = flts[0, 32]
    p0f = flts[:, :16].reshape(-1)
    p1f = flts[:, 16:32].reshape(-1)
    xs = _sc_scatter_call(x2d, dflat)
    ys = _ffn_call(te, vd, xs, w_gate, w_up, w_down)
    out = _sc_combine_call(ys, dflat, p0f, p1f)
    return out.reshape(1, S, H), aux


# speedup vs baseline: 1.0263x; 1.0263x over previous
import functools

import jax
import jax.numpy as jnp
from jax import lax
from jax.experimental import pallas as pl
from jax.experimental.pallas import tpu as pltpu
from jax.experimental.pallas import tpu_sc as plsc

S = 2048
H = 768
F = 2048
E = 8
LANES = 128
TM = 512
NT = S * 2 // TM + E
PADDED = NT * TM
NW = 32
TPW = S // NW
CH = 16


def _cumsum_rows(a):
    n = a.shape[0]
    sh = 1
    while sh < n:
        a = a + jnp.concatenate(
            [jnp.zeros((sh, a.shape[1]), a.dtype), a[: n - sh]], axis=0)
        sh *= 2
    return a


def _cumsum_lanes8(a):
    for sh in (1, 2, 4):
        a = a + jnp.concatenate(
            [jnp.zeros((a.shape[0], sh), a.dtype), a[:, : a.shape[1] - sh]],
            axis=1)
    return a


def _router_body(x_ref, wr_ref, ints_ref, flt_ref):
    x = x_ref[...]
    wr = wr_ref[...]
    logits = jnp.dot(x, wr, preferred_element_type=jnp.float32)
    col = lax.broadcasted_iota(jnp.int32, (S, LANES), 1)
    valid = col < E
    lg = jnp.where(valid, logits, jnp.float32(-1e30))
    m = jnp.max(lg, axis=1, keepdims=True)
    ex = jnp.where(valid, jnp.exp(lg - m), 0.0)
    probs = ex / jnp.sum(ex, axis=1, keepdims=True)

    usage = jnp.sum(probs, axis=0, keepdims=True) * (1.0 / S)
    dd = jnp.where(valid[0:1, :], usage - (1.0 / E), 0.0)
    aux = jnp.sum(dd * dd) * 0.01

    v0 = jnp.max(probs, axis=1, keepdims=True)
    i0 = jnp.min(jnp.where((probs == v0) & valid, col, LANES), axis=1,
                 keepdims=True)
    pm = jnp.where((col == i0) | ~valid, -1.0, probs)
    v1 = jnp.max(pm, axis=1, keepdims=True)
    i1 = jnp.min(jnp.where(pm == v1, col, LANES), axis=1, keepdims=True)
    sn = v0 + v1
    p0 = v0 / sn
    p1 = v1 / sn

    oc = ((col == i0) | (col == i1 + 8)).astype(jnp.float32)
    mask_lo = (col < E).astype(jnp.float32)
    mask_hi = ((col >= 8) & (col < 16)).astype(jnp.float32)
    inc = _cumsum_rows(oc)
    r = inc - oc
    tot = inc[S - 1:S, :]
    z8 = jnp.zeros((1, 8), jnp.float32)
    tot_hi = jnp.concatenate([z8, tot[:, :LANES - 8]], axis=1)
    rank0 = jnp.sum(r * oc * mask_lo, axis=1, keepdims=True)
    rank1 = jnp.sum((r + tot_hi) * oc * mask_hi, axis=1, keepdims=True)
    counts = tot + jnp.concatenate(
        [tot[:, 8:], jnp.zeros((1, 8), jnp.float32)], axis=1)
    pcnt = jnp.floor((counts + (TM - 1)) * (1.0 / TM)) * TM * mask_lo[0:1, :]
    inc_p = _cumsum_lanes8(pcnt)
    off = inc_p - pcnt
    off_hi = jnp.concatenate([z8, off[:, :LANES - 8]], axis=1)
    base0 = jnp.sum(oc * mask_lo * off, axis=1, keepdims=True)
    base1 = jnp.sum(oc * mask_hi * off_hi, axis=1, keepdims=True)
    dest0 = (base0 + rank0).astype(jnp.int32)
    dest1 = (base1 + rank1).astype(jnp.int32)

    start = (lax.broadcasted_iota(jnp.int32, (S, 1), 0) * TM).astype(jnp.float32)
    indmat = ((inc_p <= start) & valid).astype(jnp.float32)
    te = jnp.minimum(jnp.sum(indmat, axis=1, keepdims=True),
                     float(E - 1)).astype(jnp.int32)
    total = inc_p[:, E - 1:E]
    vt = (start < total).astype(jnp.int32)

    ints_ref[...] = jnp.where(
        col == 0, dest0,
        jnp.where(col == 1, dest1,
                  jnp.where(col == 2, te, jnp.where(col == 3, vt, 0))))
    flt_ref[...] = jnp.where(col < 16, p0, jnp.where(col < 32, p1, aux))


def _router_call(x2d, wrp, interpret=False):
    return pl.pallas_call(
        _router_body,
        out_shape=(jax.ShapeDtypeStruct((S, LANES), jnp.int32),
                   jax.ShapeDtypeStruct((S, LANES), jnp.float32)),
        interpret=interpret,
    )(x2d, wrp)


def _ffn_body(te_ref, vd_ref, xs_ref, wg_ref, wu_ref, wd_ref, ys_ref):
    t = pl.program_id(0)

    @pl.when(vd_ref[t] == 1)
    def _():
        xb = xs_ref[...].astype(jnp.bfloat16)
        wg = wg_ref[0].astype(jnp.bfloat16)
        wu = wu_ref[0].astype(jnp.bfloat16)
        g = jnp.dot(xb, wg, preferred_element_type=jnp.float32)
        u = jnp.dot(xb, wu, preferred_element_type=jnp.float32)
        h = ((g * jax.nn.sigmoid(g)) * u).astype(jnp.bfloat16)
        wd = wd_ref[0].astype(jnp.bfloat16)
        ys_ref[...] = jnp.dot(h, wd, preferred_element_type=jnp.float32)


def _ffn_call(te, vd, xs, w_gate, w_up, w_down, interpret=False):
    grid_spec = pltpu.PrefetchScalarGridSpec(
        num_scalar_prefetch=2,
        grid=(NT,),
        in_specs=[
            pl.BlockSpec((TM, H), lambda t, te, vd: (t * vd[t], 0)),
            pl.BlockSpec((1, H, F), lambda t, te, vd: (te[t], 0, 0)),
            pl.BlockSpec((1, H, F), lambda t, te, vd: (te[t], 0, 0)),
            pl.BlockSpec((1, F, H), lambda t, te, vd: (te[t], 0, 0)),
        ],
        out_specs=pl.BlockSpec((TM, H), lambda t, te, vd: (t, 0)),
    )
    return pl.pallas_call(
        _ffn_body,
        grid_spec=grid_spec,
        out_shape=jax.ShapeDtypeStruct((PADDED, H), jnp.float32),
        interpret=interpret,
    )(te, vd, xs, w_gate, w_up, w_down)


def _sc_scatter_call(x2d, dest2):
    mesh = plsc.VectorSubcoreMesh(core_axis_name="c", subcore_axis_name="s")

    @functools.partial(
        pl.kernel,
        out_type=jax.ShapeDtypeStruct((PADDED, H), jnp.float32),
        mesh=mesh,
        scratch_types=[
            pltpu.VMEM((TPW, H), jnp.float32),
            pltpu.VMEM((TPW,), jnp.int32),
            pltpu.VMEM((TPW,), jnp.int32),
            pltpu.SemaphoreType.DMA,
        ],
    )
    def k(x_hbm, d_hbm, xs_hbm, xbuf, i0buf, i1buf, sem):
        wid = lax.axis_index("s") * 2 + lax.axis_index("c")
        base = wid * TPW
        pltpu.sync_copy(x_hbm.at[pl.ds(base, TPW)], xbuf)
        pltpu.sync_copy(d_hbm.at[pl.ds(base, TPW)], i0buf)
        pltpu.sync_copy(d_hbm.at[pl.ds(S + base, TPW)], i1buf)
        pltpu.async_copy(xbuf, xs_hbm.at[i0buf], sem).wait()
        pltpu.async_copy(xbuf, xs_hbm.at[i1buf], sem).wait()

    return k(x2d, dest2)


def _sc_combine_call(ys, dest2, probs0, probs1):
    mesh = plsc.VectorSubcoreMesh(core_axis_name="c", subcore_axis_name="s")

    nchunk = TPW // CH

    @functools.partial(
        pl.kernel,
        out_type=jax.ShapeDtypeStruct((S, H), jnp.float32),
        mesh=mesh,
        scratch_types=[
            pltpu.VMEM((2, CH, H), jnp.float32),
            pltpu.VMEM((2, CH, H), jnp.float32),
            pltpu.VMEM((2, CH, H), jnp.float32),
            pltpu.VMEM((2, CH), jnp.int32),
            pltpu.VMEM((2, CH), jnp.int32),
            pltpu.VMEM((TPW * 16,), jnp.float32),
            pltpu.VMEM((TPW * 16,), jnp.float32),
            pltpu.SemaphoreType.DMA,
            pltpu.SemaphoreType.DMA,
        ],
    )
    def k(ys_hbm, d_hbm, p0_hbm, p1_hbm, out_hbm,
          y0buf, y1buf, obuf, i0buf, i1buf, p0buf, p1buf, gsem, ssem):
        wid = lax.axis_index("s") * 2 + lax.axis_index("c")
        base = wid * TPW
        pltpu.sync_copy(p0_hbm.at[pl.ds(base * 16, TPW * 16)], p0buf)
        pltpu.sync_copy(p1_hbm.at[pl.ds(base * 16, TPW * 16)], p1buf)

        def start(c):
            s = c % 2
            cb = base + c * CH
            pltpu.sync_copy(d_hbm.at[pl.ds(cb, CH)], i0buf.at[s])
            pltpu.sync_copy(d_hbm.at[pl.ds(S + cb, CH)], i1buf.at[s])
            return (pltpu.async_copy(ys_hbm.at[i0buf.at[s]], y0buf.at[s], gsem),
                    pltpu.async_copy(ys_hbm.at[i1buf.at[s]], y1buf.at[s], gsem))

        gets = {0: start(0)}
        puts = {}
        for c in range(nchunk):
            s = c % 2
            if c + 1 < nchunk:
                gets[c + 1] = start(c + 1)
            for cp in gets.pop(c):
                cp.wait()
            if c >= 2:
                puts.pop(c - 2).wait()

            def body(i, _):
                pa = p0buf[pl.ds((c * CH + i) * 16, 16)]
                pb = p1buf[pl.ds((c * CH + i) * 16, 16)]
                for j in range(H // 16):
                    sl = pl.ds(j * 16, 16)
                    obuf[s, i, sl] = y0buf[s, i, sl] * pa + y1buf[s, i, sl] * pb
                return 0

            lax.fori_loop(0, CH, body, 0)
            puts[c] = pltpu.async_copy(
                obuf.at[s], out_hbm.at[pl.ds(base + c * CH, CH)], ssem)
        for c in sorted(puts):
            puts.pop(c).wait()

    return k(ys, dest2, probs0, probs1)


def kernel(x, w_router, w_gate, w_up, w_down):
    x2d = x.reshape(S, H)
    wrp = jnp.zeros((H, LANES), jnp.float32).at[:, :E].set(w_router.T)
    ints, flts = _router_call(x2d, wrp)
    dflat = jnp.concatenate([ints[:, 0], ints[:, 1]])
    te = ints[:NT, 2]
    vd = ints[:NT, 3]
    aux = flts[0, 32]
    p0f = flts[:, :16].reshape(-1)
    p1f = flts[:, 16:32].reshape(-1)
    xs = _sc_scatter_call(x2d, dflat)
    ys = _ffn_call(te, vd, xs, w_gate, w_up, w_down)
    out = _sc_combine_call(ys, dflat, p0f, p1f)
    return out.reshape(1, S, H), aux


# speedup vs baseline: 1.0529x; 1.0260x over previous
import functools

import jax
import jax.numpy as jnp
from jax import lax
from jax.experimental import pallas as pl
from jax.experimental.pallas import tpu as pltpu
from jax.experimental.pallas import tpu_sc as plsc

S = 2048
H = 768
F = 2048
E = 8
LANES = 128
TM = 512
NT = S * 2 // TM + E
PADDED = NT * TM
NW = 32
TPW = S // NW
CH = 16


def _cumsum_rows(a):
    n = a.shape[0]
    sh = 1
    while sh < n:
        a = a + jnp.concatenate(
            [jnp.zeros((sh, a.shape[1]), a.dtype), a[: n - sh]], axis=0)
        sh *= 2
    return a


def _cumsum_lanes8(a):
    for sh in (1, 2, 4):
        a = a + jnp.concatenate(
            [jnp.zeros((a.shape[0], sh), a.dtype), a[:, : a.shape[1] - sh]],
            axis=1)
    return a


def _router_body(x_ref, wr_ref, ints_ref, flt_ref):
    x = x_ref[...]
    wr = wr_ref[...]
    lg8 = lax.dot_general(x, wr, (((1,), (1,)), ((), ())),
                          preferred_element_type=jnp.float32)
    logits = jnp.concatenate(
        [lg8, jnp.zeros((S, LANES - E), jnp.float32)], axis=1)
    col = lax.broadcasted_iota(jnp.int32, (S, LANES), 1)
    valid = col < E
    lg = jnp.where(valid, logits, jnp.float32(-1e30))
    m = jnp.max(lg, axis=1, keepdims=True)
    ex = jnp.where(valid, jnp.exp(lg - m), 0.0)
    probs = ex / jnp.sum(ex, axis=1, keepdims=True)

    usage = jnp.sum(probs, axis=0, keepdims=True) * (1.0 / S)
    dd = jnp.where(valid[0:1, :], usage - (1.0 / E), 0.0)
    aux = jnp.sum(dd * dd) * 0.01

    v0 = jnp.max(probs, axis=1, keepdims=True)
    i0 = jnp.min(jnp.where((probs == v0) & valid, col, LANES), axis=1,
                 keepdims=True)
    pm = jnp.where((col == i0) | ~valid, -1.0, probs)
    v1 = jnp.max(pm, axis=1, keepdims=True)
    i1 = jnp.min(jnp.where(pm == v1, col, LANES), axis=1, keepdims=True)
    sn = v0 + v1
    p0 = v0 / sn
    p1 = v1 / sn

    oc = ((col == i0) | (col == i1 + 8)).astype(jnp.float32)
    mask_lo = (col < E).astype(jnp.float32)
    mask_hi = ((col >= 8) & (col < 16)).astype(jnp.float32)
    inc = _cumsum_rows(oc)
    r = inc - oc
    tot = inc[S - 1:S, :]
    z8 = jnp.zeros((1, 8), jnp.float32)
    tot_hi = jnp.concatenate([z8, tot[:, :LANES - 8]], axis=1)
    rank0 = jnp.sum(r * oc * mask_lo, axis=1, keepdims=True)
    rank1 = jnp.sum((r + tot_hi) * oc * mask_hi, axis=1, keepdims=True)
    counts = tot + jnp.concatenate(
        [tot[:, 8:], jnp.zeros((1, 8), jnp.float32)], axis=1)
    pcnt = jnp.floor((counts + (TM - 1)) * (1.0 / TM)) * TM * mask_lo[0:1, :]
    inc_p = _cumsum_lanes8(pcnt)
    off = inc_p - pcnt
    off_hi = jnp.concatenate([z8, off[:, :LANES - 8]], axis=1)
    base0 = jnp.sum(oc * mask_lo * off, axis=1, keepdims=True)
    base1 = jnp.sum(oc * mask_hi * off_hi, axis=1, keepdims=True)
    dest0 = (base0 + rank0).astype(jnp.int32)
    dest1 = (base1 + rank1).astype(jnp.int32)

    start = (lax.broadcasted_iota(jnp.int32, (S, 1), 0) * TM).astype(jnp.float32)
    indmat = ((inc_p <= start) & valid).astype(jnp.float32)
    te = jnp.minimum(jnp.sum(indmat, axis=1, keepdims=True),
                     float(E - 1)).astype(jnp.int32)
    total = inc_p[:, E - 1:E]
    vt = (start < total).astype(jnp.int32)

    ints_ref[...] = jnp.where(
        col == 0, dest0,
        jnp.where(col == 1, dest1,
                  jnp.where(col == 2, te, jnp.where(col == 3, vt, 0))))
    flt_ref[...] = jnp.where(col < 16, p0, jnp.where(col < 32, p1, aux))


def _router_call(x2d, w_router, interpret=False):
    return pl.pallas_call(
        _router_body,
        out_shape=(jax.ShapeDtypeStruct((S, LANES), jnp.int32),
                   jax.ShapeDtypeStruct((S, LANES), jnp.float32)),
        interpret=interpret,
    )(x2d, w_router)


def _ffn_body(te_ref, vd_ref, xs_ref, wg_ref, wu_ref, wd_ref, ys_ref):
    t = pl.program_id(0)

    @pl.when(vd_ref[t] == 1)
    def _():
        xb = xs_ref[...].astype(jnp.bfloat16)
        wg = wg_ref[0].astype(jnp.bfloat16)
        wu = wu_ref[0].astype(jnp.bfloat16)
        g = jnp.dot(xb, wg, preferred_element_type=jnp.float32)
        u = jnp.dot(xb, wu, preferred_element_type=jnp.float32)
        h = ((g * jax.nn.sigmoid(g)) * u).astype(jnp.bfloat16)
        wd = wd_ref[0].astype(jnp.bfloat16)
        ys_ref[...] = jnp.dot(h, wd, preferred_element_type=jnp.float32)


def _ffn_call(te, vd, xs, w_gate, w_up, w_down, interpret=False):
    grid_spec = pltpu.PrefetchScalarGridSpec(
        num_scalar_prefetch=2,
        grid=(NT,),
        in_specs=[
            pl.BlockSpec((TM, H), lambda t, te, vd: (t * vd[t], 0)),
            pl.BlockSpec((1, H, F), lambda t, te, vd: (te[t], 0, 0)),
            pl.BlockSpec((1, H, F), lambda t, te, vd: (te[t], 0, 0)),
            pl.BlockSpec((1, F, H), lambda t, te, vd: (te[t], 0, 0)),
        ],
        out_specs=pl.BlockSpec((TM, H), lambda t, te, vd: (t, 0)),
    )
    return pl.pallas_call(
        _ffn_body,
        grid_spec=grid_spec,
        out_shape=jax.ShapeDtypeStruct((PADDED, H), jnp.float32),
        interpret=interpret,
    )(te, vd, xs, w_gate, w_up, w_down)


def _sc_scatter_call(x2d, dest2):
    mesh = plsc.VectorSubcoreMesh(core_axis_name="c", subcore_axis_name="s")

    @functools.partial(
        pl.kernel,
        out_type=jax.ShapeDtypeStruct((PADDED, H), jnp.float32),
        mesh=mesh,
        scratch_types=[
            pltpu.VMEM((TPW, H), jnp.float32),
            pltpu.VMEM((TPW,), jnp.int32),
            pltpu.VMEM((TPW,), jnp.int32),
            pltpu.SemaphoreType.DMA,
        ],
    )
    def k(x_hbm, d_hbm, xs_hbm, xbuf, i0buf, i1buf, sem):
        wid = lax.axis_index("s") * 2 + lax.axis_index("c")
        base = wid * TPW
        cpx = pltpu.async_copy(x_hbm.at[pl.ds(base, TPW)], xbuf, sem)
        pltpu.sync_copy(d_hbm.at[pl.ds(base, TPW)], i0buf)
        pltpu.sync_copy(d_hbm.at[pl.ds(S + base, TPW)], i1buf)
        cpx.wait()
        c0 = pltpu.async_copy(xbuf, xs_hbm.at[i0buf], sem)
        c1 = pltpu.async_copy(xbuf, xs_hbm.at[i1buf], sem)
        c0.wait()
        c1.wait()

    return k(x2d, dest2)


def _sc_combine_call(ys, dest2, probs0, probs1):
    mesh = plsc.VectorSubcoreMesh(core_axis_name="c", subcore_axis_name="s")

    nchunk = TPW // CH

    @functools.partial(
        pl.kernel,
        out_type=jax.ShapeDtypeStruct((S, H), jnp.float32),
        mesh=mesh,
        scratch_types=[
            pltpu.VMEM((2, CH, H), jnp.float32),
            pltpu.VMEM((2, CH, H), jnp.float32),
            pltpu.VMEM((2, CH, H), jnp.float32),
            pltpu.VMEM((2, CH), jnp.int32),
            pltpu.VMEM((2, CH), jnp.int32),
            pltpu.VMEM((TPW * 16,), jnp.float32),
            pltpu.VMEM((TPW * 16,), jnp.float32),
            pltpu.SemaphoreType.DMA,
            pltpu.SemaphoreType.DMA,
        ],
    )
    def k(ys_hbm, d_hbm, p0_hbm, p1_hbm, out_hbm,
          y0buf, y1buf, obuf, i0buf, i1buf, p0buf, p1buf, gsem, ssem):
        wid = lax.axis_index("s") * 2 + lax.axis_index("c")
        base = wid * TPW
        pltpu.sync_copy(p0_hbm.at[pl.ds(base * 16, TPW * 16)], p0buf)
        pltpu.sync_copy(p1_hbm.at[pl.ds(base * 16, TPW * 16)], p1buf)

        def start(c):
            s = c % 2
            cb = base + c * CH
            pltpu.sync_copy(d_hbm.at[pl.ds(cb, CH)], i0buf.at[s])
            pltpu.sync_copy(d_hbm.at[pl.ds(S + cb, CH)], i1buf.at[s])
            return (pltpu.async_copy(ys_hbm.at[i0buf.at[s]], y0buf.at[s], gsem),
                    pltpu.async_copy(ys_hbm.at[i1buf.at[s]], y1buf.at[s], gsem))

        gets = {0: start(0)}
        puts = {}
        for c in range(nchunk):
            s = c % 2
            if c + 1 < nchunk:
                gets[c + 1] = start(c + 1)
            for cp in gets.pop(c):
                cp.wait()
            if c >= 2:
                puts.pop(c - 2).wait()

            def body(i, _):
                pa = p0buf[pl.ds((c * CH + i) * 16, 16)]
                pb = p1buf[pl.ds((c * CH + i) * 16, 16)]
                for j in range(H // 16):
                    sl = pl.ds(j * 16, 16)
                    obuf[s, i, sl] = y0buf[s, i, sl] * pa + y1buf[s, i, sl] * pb
                return 0

            lax.fori_loop(0, CH, body, 0)
            puts[c] = pltpu.async_copy(
                obuf.at[s], out_hbm.at[pl.ds(base + c * CH, CH)], ssem)
        for c in sorted(puts):
            puts.pop(c).wait()

    return k(ys, dest2, probs0, probs1)


def kernel(x, w_router, w_gate, w_up, w_down):
    x2d = x.reshape(S, H)
    ints, flts = _router_call(x2d, w_router)
    dflat = jnp.concatenate([ints[:, 0], ints[:, 1]])
    te = ints[:NT, 2]
    vd = ints[:NT, 3]
    aux = flts[0, 32]
    p0f = flts[:, :16].reshape(-1)
    p1f = flts[:, 16:32].reshape(-1)
    xs = _sc_scatter_call(x2d, dflat)
    ys = _ffn_call(te, vd, xs, w_gate, w_up, w_down)
    out = _sc_combine_call(ys, dflat, p0f, p1f)
    return out.reshape(1, S, H), aux
